# baseline (speedup 1.0000x reference)
import dataclasses
import functools

import jax
import jax.numpy as jnp
from jax import lax
from jax.experimental import pallas as pl
from jax.experimental.pallas import tpu as pltpu
from jax.experimental.pallas import tpu_sc as plsc

_B, _NBOX, _NNEG = 4, 50, 100
_H, _W = 248, 216
_HW = _H * _W
_NPOS = _B * _NBOX
_NBG = _B * _NNEG
_NPOS_PAD = 208
_ALPHA = 0.25
_BETA_LOC = 2.0
_LN2 = 0.6931471805599453


def _ln(p):
    bits = lax.bitcast_convert_type(p, jnp.int32)
    e = jnp.right_shift(bits, 23) - 127
    m = lax.bitcast_convert_type(
        jnp.bitwise_or(jnp.bitwise_and(bits, 0x007FFFFF), 0x3F800000),
        jnp.float32)
    t = (m - 1.0) / (m + 1.0)
    t2 = t * t
    ln_m = t * (2.0 + t2 * (2.0 / 3.0 + t2 * (2.0 / 5.0
                + t2 * (2.0 / 7.0 + t2 * (2.0 / 9.0)))))
    return e.astype(jnp.float32) * _LN2 + ln_m


def _focal(p):
    one_m = 1.0 - p
    return -_ln(p) * (_ALPHA * one_m * one_m)


def _huber(x):
    ax = jnp.abs(x)
    return jnp.where(ax < 1.0, 0.5 * x * x, ax - 0.5)


_mesh = plsc.VectorSubcoreMesh(core_axis_name="c", subcore_axis_name="s")

_cp = pltpu.CompilerParams()
if "needs_layout_passes" in pltpu.CompilerParams.__dataclass_fields__:
    _cp = dataclasses.replace(_cp, needs_layout_passes=False)


@functools.partial(
    pl.kernel,
    out_type=jax.ShapeDtypeStruct((16,), jnp.float32),
    mesh=_mesh,
    compiler_params=_cp,
    scratch_types=[
        pltpu.VMEM((2 * _NPOS,), jnp.int32),
        pltpu.VMEM((3 * _NBG,), jnp.int32),
        pltpu.VMEM((4 * _NPOS,), jnp.float32),
        pltpu.VMEM((16,), jnp.float32),
        pltpu.VMEM((_NPOS_PAD,), jnp.int32),
        pltpu.VMEM((_NBG,), jnp.int32),
        pltpu.VMEM((_NPOS_PAD,), jnp.float32),
        pltpu.VMEM((_NPOS_PAD,), jnp.float32),
        pltpu.VMEM((_NPOS_PAD,), jnp.float32),
        pltpu.VMEM((_NBG,), jnp.float32),
        pltpu.VMEM((16,), jnp.float32),
        pltpu.SemaphoreType.DMA,
    ],
)
def _loss_kernel(rt_hbm, ct_hbm, gt_hbm, consts_hbm,
                 locx_hbm, locy_hbm, car_hbm, bg_hbm, out_hbm,
                 rt_v, ct_v, gt_v, consts_v, pidx_v, bidx_v,
                 lx_v, ly_v, car_v, bgv_v, out_v, sem):
    cid = lax.axis_index("c")
    sid = lax.axis_index("s")

    @pl.when(jnp.logical_and(cid == 0, sid == 0))
    def _():
        pltpu.sync_copy(rt_hbm, rt_v)
        pltpu.sync_copy(ct_hbm, ct_v)
        pltpu.sync_copy(gt_hbm, gt_v)
        pltpu.sync_copy(consts_hbm, consts_v)

        lanes = lax.iota(jnp.int32, 16)

        for i in range(_NPOS_PAD // 16):
            p = lanes + (i * 16)
            valid = p < _NPOS
            psafe = jnp.where(valid, p, 0)
            x = plsc.load_gather(rt_v, [psafe * 2])
            y = plsc.load_gather(rt_v, [psafe * 2 + 1])
            b = (jnp.where(p >= _NBOX, 1, 0)
                 + jnp.where(p >= 2 * _NBOX, 1, 0)
                 + jnp.where(p >= 3 * _NBOX, 1, 0))
            base = b * _HW + y * _W + x
            pidx_v[pl.ds(i * 16, 16)] = jnp.where(valid, base, 0)

        for i in range(_NBG // 16):
            q = lanes + (i * 16)
            bx = plsc.load_gather(ct_v, [q * 3 + 1])
            by = plsc.load_gather(ct_v, [q * 3 + 2])
            b = (jnp.where(q >= _NNEG, 1, 0)
                 + jnp.where(q >= 2 * _NNEG, 1, 0)
                 + jnp.where(q >= 3 * _NNEG, 1, 0))
            bidx_v[pl.ds(i * 16, 16)] = b * _HW + by * _W + bx

        copies = []
        for tab, idxv, dst, n in ((locx_hbm, pidx_v, lx_v, _NPOS_PAD),
                                  (locy_hbm, pidx_v, ly_v, _NPOS_PAD),
                                  (car_hbm, pidx_v, car_v, _NPOS_PAD),
                                  (bg_hbm, bidx_v, bgv_v, _NBG)):
            for off in range(0, n, 128):
                sz = min(128, n - off)
                copies.append(pltpu.async_copy(
                    tab.at[idxv.at[pl.ds(off, sz)]],
                    dst.at[pl.ds(off, sz)], sem))
        for c in copies:
            c.wait()

        inv_da = consts_v[pl.ds(0, 16)]

        sl_acc = jnp.zeros((16,), jnp.float32)
        car_acc = jnp.zeros((16,), jnp.float32)
        for i in range(_NPOS_PAD // 16):
            p = lanes + (i * 16)
            valid = p < _NPOS
            w = jnp.where(valid, 1.0, 0.0)
            psafe = jnp.where(valid, p, 0)
            g0 = plsc.load_gather(gt_v, [psafe * 4])
            g1 = plsc.load_gather(gt_v, [psafe * 4 + 1])
            g2 = plsc.load_gather(gt_v, [psafe * 4 + 2])
            g3 = plsc.load_gather(gt_v, [psafe * 4 + 3])
            x_gt = g0 + (g2 - g0) * 0.5
            y_gt = g1 - (g3 - g1) * 0.5
            dx = (x_gt - lx_v[pl.ds(i * 16, 16)]) * inv_da
            dy = (y_gt - ly_v[pl.ds(i * 16, 16)]) * inv_da
            sl_acc = sl_acc + w * (_huber(dx) + _huber(dy))
            car_acc = car_acc + w * _focal(car_v[pl.ds(i * 16, 16)])

        bg_acc = jnp.zeros((16,), jnp.float32)
        for i in range(_NBG // 16):
            bg_acc = bg_acc + _focal(bgv_v[pl.ds(i * 16, 16)])

        tot = (sl_acc * (_BETA_LOC / _NPOS)
               + car_acc * (1.0 / ((_B - 1) * (_NBOX - 1)))
               + bg_acc * (1.0 / ((_B - 1) * (_NNEG - 1))))
        out_v[...] = jnp.zeros((16,), jnp.float32) + jnp.sum(tot)
        pltpu.sync_copy(out_v, out_hbm)


def kernel(regression_targets, classification_targets_dict, gt_boxes_tensor,
           loc, size, clf, occupancy, angle, heading, anchor):
    rt = regression_targets.reshape(-1).astype(jnp.int32)
    ct = classification_targets_dict.reshape(-1).astype(jnp.int32)
    gt = gt_boxes_tensor.reshape(-1).astype(jnp.float32)
    locx = loc[:, 0, 0].reshape(-1)
    locy = loc[:, 0, 1].reshape(-1)
    car = clf[:, 0, 1].reshape(-1)
    bg = clf[:, 0, 0].reshape(-1)
    a0 = anchor[0].astype(jnp.float32)
    a1 = anchor[1].astype(jnp.float32)
    inv_da = 1.0 / jnp.sqrt(a0 * a0 + a1 * a1)
    consts = jnp.broadcast_to(inv_da, (16,))
    out = _loss_kernel(rt, ct, gt, consts, locx, locy, car, bg)
    return out[0]


# speedup vs baseline: 1.1019x; 1.1019x over previous
import dataclasses
import functools

import jax
import jax.numpy as jnp
from jax import lax
from jax.experimental import pallas as pl
from jax.experimental.pallas import tpu as pltpu
from jax.experimental.pallas import tpu_sc as plsc

_B, _NBOX, _NNEG = 4, 50, 100
_H, _W = 248, 216
_HW = _H * _W
_PLANE = _B * _HW
_NPOS = _B * _NBOX
_NBG = _B * _NNEG
_NPOS_PAD = 208
_NIDX = 3 * _NPOS_PAD + _NBG
_OFF_RT = 0
_OFF_CT = 400
_OFF_GT = 1600
_OFF_INV = 2400
_NPACK = 2416
_ALPHA = 0.25
_BETA_LOC = 2.0
_LN2 = 0.6931471805599453


def _ln(p):
    bits = lax.bitcast_convert_type(p, jnp.int32)
    e = jnp.right_shift(bits, 23) - 127
    m = lax.bitcast_convert_type(
        jnp.bitwise_or(jnp.bitwise_and(bits, 0x007FFFFF), 0x3F800000),
        jnp.float32)
    t = (m - 1.0) / (m + 1.0)
    t2 = t * t
    ln_m = t * (2.0 + t2 * (2.0 / 3.0 + t2 * (2.0 / 5.0
                + t2 * (2.0 / 7.0 + t2 * (2.0 / 9.0)))))
    return e.astype(jnp.float32) * _LN2 + ln_m


def _focal(p):
    one_m = 1.0 - p
    return -_ln(p) * (_ALPHA * one_m * one_m)


def _huber(x):
    ax = jnp.abs(x)
    return jnp.where(ax < 1.0, 0.5 * x * x, ax - 0.5)


_mesh = plsc.VectorSubcoreMesh(core_axis_name="c", subcore_axis_name="s")

_cp = pltpu.CompilerParams()
if "needs_layout_passes" in pltpu.CompilerParams.__dataclass_fields__:
    _cp = dataclasses.replace(_cp, needs_layout_passes=False)


@functools.partial(
    pl.kernel,
    out_type=jax.ShapeDtypeStruct((16,), jnp.float32),
    mesh=_mesh,
    compiler_params=_cp,
    scratch_types=[
        pltpu.VMEM((_NPACK,), jnp.int32),
        pltpu.VMEM((_NIDX,), jnp.int32),
        pltpu.VMEM((_NIDX,), jnp.float32),
        pltpu.VMEM((16,), jnp.float32),
        pltpu.SemaphoreType.DMA,
    ],
)
def _loss_kernel(pk_hbm, tab_hbm, out_hbm,
                 pk_v, idx_v, val_v, out_v, sem):
    cid = lax.axis_index("c")
    sid = lax.axis_index("s")

    @pl.when(jnp.logical_and(cid == 0, sid == 0))
    def _():
        pltpu.sync_copy(pk_hbm, pk_v)

        lanes = lax.iota(jnp.int32, 16)

        for i in range(_NPOS_PAD // 16):
            p = lanes + (i * 16)
            valid = p < _NPOS
            psafe = jnp.where(valid, p, 0)
            x = plsc.load_gather(pk_v, [psafe * 2])
            y = plsc.load_gather(pk_v, [psafe * 2 + 1])
            b = (jnp.where(p >= _NBOX, 1, 0)
                 + jnp.where(p >= 2 * _NBOX, 1, 0)
                 + jnp.where(p >= 3 * _NBOX, 1, 0))
            base = jnp.where(valid, b * _HW + y * _W + x, 0)
            idx_v[pl.ds(i * 16, 16)] = base
            idx_v[pl.ds(_NPOS_PAD + i * 16, 16)] = base + _PLANE
            idx_v[pl.ds(2 * _NPOS_PAD + i * 16, 16)] = base + 2 * _PLANE

        copies = [pltpu.async_copy(tab_hbm.at[idx_v.at[pl.ds(off, 128)]],
                                   val_v.at[pl.ds(off, 128)], sem)
                  for off in range(0, 512, 128)]

        for i in range(_NBG // 16):
            q = lanes + (i * 16)
            bx = plsc.load_gather(pk_v, [_OFF_CT + q * 3 + 1])
            by = plsc.load_gather(pk_v, [_OFF_CT + q * 3 + 2])
            b = (jnp.where(q >= _NNEG, 1, 0)
                 + jnp.where(q >= 2 * _NNEG, 1, 0)
                 + jnp.where(q >= 3 * _NNEG, 1, 0))
            idx_v[pl.ds(3 * _NPOS_PAD + i * 16, 16)] = (
                3 * _PLANE + b * _HW + by * _W + bx)

        copies += [pltpu.async_copy(tab_hbm.at[idx_v.at[pl.ds(off, 128)]],
                                    val_v.at[pl.ds(off, 128)], sem)
                   for off in range(512, _NIDX, 128)]

        inv_da = plsc.bitcast(pk_v[pl.ds(_OFF_INV, 16)], jnp.float32)

        for c in copies:
            c.wait()

        sl_acc = jnp.zeros((16,), jnp.float32)
        car_acc = jnp.zeros((16,), jnp.float32)
        for i in range(_NPOS_PAD // 16):
            p = lanes + (i * 16)
            valid = p < _NPOS
            w = jnp.where(valid, 1.0, 0.0)
            psafe = jnp.where(valid, p, 0)
            g0 = plsc.bitcast(
                plsc.load_gather(pk_v, [_OFF_GT + psafe * 4]), jnp.float32)
            g1 = plsc.bitcast(
                plsc.load_gather(pk_v, [_OFF_GT + psafe * 4 + 1]), jnp.float32)
            g2 = plsc.bitcast(
                plsc.load_gather(pk_v, [_OFF_GT + psafe * 4 + 2]), jnp.float32)
            g3 = plsc.bitcast(
                plsc.load_gather(pk_v, [_OFF_GT + psafe * 4 + 3]), jnp.float32)
            x_gt = g0 + (g2 - g0) * 0.5
            y_gt = g1 - (g3 - g1) * 0.5
            dx = (x_gt - val_v[pl.ds(i * 16, 16)]) * inv_da
            dy = (y_gt - val_v[pl.ds(_NPOS_PAD + i * 16, 16)]) * inv_da
            sl_acc = sl_acc + w * (_huber(dx) + _huber(dy))
            car_acc = car_acc + w * _focal(
                val_v[pl.ds(2 * _NPOS_PAD + i * 16, 16)])

        bg_acc = jnp.zeros((16,), jnp.float32)
        for i in range(_NBG // 16):
            bg_acc = bg_acc + _focal(val_v[pl.ds(3 * _NPOS_PAD + i * 16, 16)])

        tot = (sl_acc * (_BETA_LOC / _NPOS)
               + car_acc * (1.0 / ((_B - 1) * (_NBOX - 1)))
               + bg_acc * (1.0 / ((_B - 1) * (_NNEG - 1))))
        out_v[...] = jnp.zeros((16,), jnp.float32) + jnp.sum(tot)
        pltpu.sync_copy(out_v, out_hbm)


def kernel(regression_targets, classification_targets_dict, gt_boxes_tensor,
           loc, size, clf, occupancy, angle, heading, anchor):
    rt = regression_targets.reshape(-1).astype(jnp.int32)
    ct = classification_targets_dict.reshape(-1).astype(jnp.int32)
    gt = lax.bitcast_convert_type(
        gt_boxes_tensor.reshape(-1).astype(jnp.float32), jnp.int32)
    a0 = anchor[0].astype(jnp.float32)
    a1 = anchor[1].astype(jnp.float32)
    inv_da = 1.0 / jnp.sqrt(a0 * a0 + a1 * a1)
    inv = lax.bitcast_convert_type(
        jnp.broadcast_to(inv_da, (16,)), jnp.int32)
    packed = jnp.concatenate([rt, ct, gt, inv])
    table = jnp.stack(
        [loc[:, 0, 0], loc[:, 0, 1], clf[:, 0, 1], clf[:, 0, 0]],
        axis=0).reshape(-1)
    out = _loss_kernel(packed, table)
    return out[0]
